# baseline (speedup 1.0000x reference)
import functools

import jax
import jax.numpy as jnp
from jax import lax
from jax.experimental import pallas as pl
from jax.experimental.pallas import tpu as pltpu
from jax.experimental.pallas import tpu_sc as plsc

B, T, D = 4, 8192, 768
NC, NS = 2, 16
NW = NC * NS
T_PER_W = T // NW
TW = 64
N_CHUNK = T_PER_W // TW
CHUNK = TW * D
N_G = CHUNK // 16
UNROLL = 8

_mesh = plsc.VectorSubcoreMesh(core_axis_name="c", subcore_axis_name="s")


@functools.partial(
    pl.kernel,
    mesh=_mesh,
    out_type=jax.ShapeDtypeStruct((B * T * D,), jnp.float32),
    scratch_types=[
        pltpu.VMEM((CHUNK,), jnp.float32),
        pltpu.VMEM((CHUNK,), jnp.float32),
    ],
)
def _sc_add(x_hbm, p_hbm, out_hbm, x_v, p_v):
    c = lax.axis_index("c")
    s = lax.axis_index("s")
    wid = s * NC + c
    t_base = wid * T_PER_W
    for ci in range(N_CHUNK):
        t0 = t_base + ci * TW
        pltpu.sync_copy(p_hbm.at[pl.ds(t0 * D, CHUNK)], p_v)
        for b in range(B):
            r0 = (b * T + t0) * D
            pltpu.sync_copy(x_hbm.at[pl.ds(r0, CHUNK)], x_v)

            def body(j, carry):
                base = j * (16 * UNROLL)
                for u in range(UNROLL):
                    off = base + u * 16
                    x_v[pl.ds(off, 16)] = x_v[pl.ds(off, 16)] + p_v[pl.ds(off, 16)]
                return carry

            lax.fori_loop(0, N_G // UNROLL, body, 0)
            pltpu.sync_copy(x_v, out_hbm.at[pl.ds(r0, CHUNK)])


def kernel(encoded_tokens, pos_table):
    b, t, d = encoded_tokens.shape
    x = encoded_tokens.reshape(b * t * d)
    p = pos_table.reshape(t * d)
    out = _sc_add(x, p)
    return out.reshape(b, t, d)


# speedup vs baseline: 1.1572x; 1.1572x over previous
import functools

import jax
import jax.numpy as jnp
from jax import lax
from jax.experimental import pallas as pl
from jax.experimental.pallas import tpu as pltpu
from jax.experimental.pallas import tpu_sc as plsc

B, T, D = 4, 8192, 768
NC, NS = 2, 16
NW = NC * NS
T_PER_W = T // NW
TW = 32
N_CHUNK = T_PER_W // TW
CHUNK = TW * D
N_G = CHUNK // 16
UNROLL = 8
NBUF = 3
STEPS = N_CHUNK * B

_mesh = plsc.VectorSubcoreMesh(core_axis_name="c", subcore_axis_name="s")


@functools.partial(
    pl.kernel,
    mesh=_mesh,
    out_type=jax.ShapeDtypeStruct((B * T * D,), jnp.float32),
    scratch_types=[
        pltpu.VMEM((CHUNK,), jnp.float32),
        pltpu.VMEM((CHUNK,), jnp.float32),
        pltpu.VMEM((CHUNK,), jnp.float32),
        pltpu.VMEM((CHUNK,), jnp.float32),
        pltpu.SemaphoreType.DMA,
        pltpu.SemaphoreType.DMA,
        pltpu.SemaphoreType.DMA,
        pltpu.SemaphoreType.DMA,
        pltpu.SemaphoreType.DMA,
        pltpu.SemaphoreType.DMA,
        pltpu.SemaphoreType.DMA,
    ],
)
def _sc_add(x_hbm, p_hbm, out_hbm, p_v, xb0, xb1, xb2,
            psem, l0, l1, l2, s0, s1, s2):
    xb = (xb0, xb1, xb2)
    lsem = (l0, l1, l2)
    ssem = (s0, s1, s2)
    c = lax.axis_index("c")
    s = lax.axis_index("s")
    wid = s * NC + c
    t_base = wid * T_PER_W

    def x_off(step):
        ci, b = step // B, step % B
        return (b * T + t_base + ci * TW) * D

    lh = [None] * STEPS
    sh = [None] * STEPS
    ph = None

    lh[0] = pltpu.async_copy(x_hbm.at[pl.ds(x_off(0), CHUNK)], xb[0], lsem[0])

    for step in range(STEPS):
        k = step % NBUF
        b = step % B
        if b == 0:
            t0 = t_base + (step // B) * TW
            ph = pltpu.async_copy(p_hbm.at[pl.ds(t0 * D, CHUNK)], p_v, psem)
        nxt = step + 1
        if nxt < STEPS:
            k1 = nxt % NBUF
            if nxt >= NBUF:
                sh[nxt - NBUF].wait()
            lh[nxt] = pltpu.async_copy(
                x_hbm.at[pl.ds(x_off(nxt), CHUNK)], xb[k1], lsem[k1])
        lh[step].wait()
        if b == 0:
            ph.wait()

        def body(j, carry, k=k):
            base = j * (16 * UNROLL)
            for u in range(UNROLL):
                off = base + u * 16
                plsc.addupdate(xb[k].at[pl.ds(off, 16)], p_v[pl.ds(off, 16)])
            return carry

        lax.fori_loop(0, N_G // UNROLL, body, 0)
        sh[step] = pltpu.async_copy(
            xb[k], out_hbm.at[pl.ds(x_off(step), CHUNK)], ssem[k])

    for step in range(STEPS - NBUF, STEPS):
        sh[step].wait()


def kernel(encoded_tokens, pos_table):
    b, t, d = encoded_tokens.shape
    x = encoded_tokens.reshape(b * t * d)
    p = pos_table.reshape(t * d)
    out = _sc_add(x, p)
    return out.reshape(b, t, d)


# speedup vs baseline: 5.3172x; 4.5950x over previous
import jax
import jax.numpy as jnp
from jax.experimental import pallas as pl


def _add_kernel(x_ref, p_ref, o_ref):
    o_ref[...] = x_ref[...] + p_ref[...][None, :, :]


def kernel(encoded_tokens, pos_table):
    B, T, D = encoded_tokens.shape
    Tb = 256
    return pl.pallas_call(
        _add_kernel,
        grid=(T // Tb,),
        in_specs=[
            pl.BlockSpec((B, Tb, D), lambda i: (0, i, 0)),
            pl.BlockSpec((Tb, D), lambda i: (i, 0)),
        ],
        out_specs=pl.BlockSpec((B, Tb, D), lambda i: (0, i, 0)),
        out_shape=jax.ShapeDtypeStruct((B, T, D), jnp.float32),
    )(encoded_tokens, pos_table)


# speedup vs baseline: 5.4698x; 1.0287x over previous
import jax
import jax.numpy as jnp
from jax.experimental import pallas as pl


def _add_kernel(x_ref, p_ref, o_ref):
    o_ref[...] = x_ref[...] + p_ref[...][None, :, :]


def kernel(encoded_tokens, pos_table):
    B, T, D = encoded_tokens.shape
    Tb = 512
    return pl.pallas_call(
        _add_kernel,
        grid=(T // Tb,),
        in_specs=[
            pl.BlockSpec((B, Tb, D), lambda i: (0, i, 0)),
            pl.BlockSpec((Tb, D), lambda i: (i, 0)),
        ],
        out_specs=pl.BlockSpec((B, Tb, D), lambda i: (0, i, 0)),
        out_shape=jax.ShapeDtypeStruct((B, T, D), jnp.float32),
    )(encoded_tokens, pos_table)
